# Initial kernel scaffold; baseline (speedup 1.0000x reference)
import jax
import jax.numpy as jnp
from jax.experimental import pallas as pl


def kernel(observations, embedding_weight, position_code):
    raise NotImplementedError("write your pallas kernel here")


# speedup vs baseline: 1.1020x; 1.1020x over previous
import functools

import jax
import jax.numpy as jnp
from jax import lax
from jax.experimental import pallas as pl
from jax.experimental.pallas import tpu as pltpu
from jax.experimental.pallas import tpu_sc as plsc

BATCH = 16384
TOKENS = 50
DIM = 64
LANES = 16
VPR = DIM // LANES
ROWS = BATCH * TOKENS
NW = 32
G_ROWS = 100
G_PAD = 104
G_PER_CHUNK = 8
CHUNK_ROWS = G_ROWS * G_PER_CHUNK
NUM_CHUNKS = ROWS // CHUNK_ROWS
CHUNKS_PER_W = NUM_CHUNKS // NW

_mesh = plsc.VectorSubcoreMesh(core_axis_name="c", subcore_axis_name="s")


@functools.partial(
    pl.kernel,
    mesh=_mesh,
    out_type=jax.ShapeDtypeStruct(
        (NUM_CHUNKS, G_PER_CHUNK, G_ROWS, DIM), jnp.float32),
    scratch_types=[
        pltpu.VMEM((G_PER_CHUNK, G_PAD), jnp.int32),
        pltpu.VMEM((G_PER_CHUNK, G_PAD, DIM), jnp.float32),
        pltpu.VMEM((TOKENS, DIM), jnp.float32),
        pltpu.SemaphoreType.DMA,
    ],
    compiler_params=pltpu.CompilerParams(use_tc_tiling_on_sc=False),
)
def _emb_kernel(idx_hbm, table_hbm, pos_hbm, out_hbm,
                idx_v, rows_v, pos_v, gsem):
    wid = lax.axis_index("s") * 2 + lax.axis_index("c")
    pltpu.sync_copy(pos_hbm, pos_v)

    def chunk_body(c, carry):
        cg = wid * CHUNKS_PER_W + c
        pltpu.sync_copy(idx_hbm.at[cg], idx_v)
        handles = [
            pltpu.async_copy(
                table_hbm.at[idx_v.at[j]],
                rows_v.at[j], gsem)
            for j in range(G_PER_CHUNK)
        ]
        for h in handles:
            h.wait()

        def add_body(r, carry2):
            pv = [pos_v[r, pl.ds(k * LANES, LANES)] for k in range(VPR)]
            for j in range(G_PER_CHUNK):
                for half in range(G_ROWS // TOKENS):
                    row = half * TOKENS + r
                    for k in range(VPR):
                        sl = pl.ds(k * LANES, LANES)
                        rows_v[j, row, sl] = rows_v[j, row, sl] + pv[k]
            return carry2

        lax.fori_loop(0, TOKENS, add_body, 0)
        for j in range(G_PER_CHUNK):
            pltpu.sync_copy(rows_v.at[j, pl.ds(0, G_ROWS)],
                            out_hbm.at[cg, j])
        return carry

    lax.fori_loop(0, CHUNKS_PER_W, chunk_body, 0)


def kernel(observations, embedding_weight, position_code):
    idx = observations.astype(jnp.int32).reshape(
        NUM_CHUNKS, G_PER_CHUNK, G_ROWS)
    idx = jnp.pad(idx, ((0, 0), (0, 0), (0, G_PAD - G_ROWS)))
    pos = position_code.reshape(TOKENS, DIM)
    out = _emb_kernel(idx, embedding_weight, pos)
    return out.reshape(BATCH, TOKENS, DIM)
